# baseline (speedup 1.0000x reference)
import functools

import jax
import jax.numpy as jnp
from jax import lax
from jax.experimental import pallas as pl
from jax.experimental.pallas import tpu as pltpu
from jax.experimental.pallas import tpu_sc as plsc

_N = 10000
_N_SEG = 64
_BM = 200

_TILES = 32
_CHUNK = 128
_CPT = 3
_ROWS_PER_TILE = _CHUNK * _CPT
_N_PAD = _TILES * _ROWS_PER_TILE


def _layer1_body(x_ref, w1_ref, adj_ref, b_ref, wn_ref, o_ref, adj_q_ref,
                 y1_ref):
    @pl.when(pl.program_id(0) == 0)
    def _():
        y1_ref[...] = jnp.dot(
            x_ref[...], w1_ref[...],
            preferred_element_type=jnp.float32).astype(jnp.bfloat16)

    a = adj_ref[...]
    adj_q_ref[...] = jnp.round(a * 127.0).astype(jnp.int8)
    acc = jnp.dot(a.astype(jnp.bfloat16), y1_ref[...],
                  preferred_element_type=jnp.float32)
    h = jnp.maximum(acc + b_ref[...], 0.0).astype(jnp.bfloat16)
    o_ref[...] = (jnp.dot(h, wn_ref[...], preferred_element_type=jnp.float32)
                  * (1.0 / 127.0)).astype(jnp.bfloat16)


def _layer1(x_in, W1, adj, b, wn):
    n = adj.shape[0]
    d = x_in.shape[1]
    kdim = W1.shape[1]
    ow = wn.shape[1]
    return pl.pallas_call(
        _layer1_body,
        grid=(n // _BM,),
        in_specs=[
            pl.BlockSpec((n, d), lambda i: (0, 0)),
            pl.BlockSpec((d, kdim), lambda i: (0, 0)),
            pl.BlockSpec((_BM, n), lambda i: (i, 0)),
            pl.BlockSpec((1, kdim), lambda i: (0, 0)),
            pl.BlockSpec((kdim, ow), lambda i: (0, 0)),
        ],
        out_specs=[
            pl.BlockSpec((_BM, ow), lambda i: (i, 0)),
            pl.BlockSpec((_BM, n), lambda i: (i, 0)),
        ],
        out_shape=[
            jax.ShapeDtypeStruct((n, ow), jnp.bfloat16),
            jax.ShapeDtypeStruct((n, n), jnp.int8),
        ],
        scratch_shapes=[pltpu.VMEM((n, kdim), jnp.bfloat16)],
        compiler_params=pltpu.CompilerParams(
            dimension_semantics=("arbitrary",)),
    )(x_in, W1, adj, b, wn)


_BM2 = 1000


def _layer_body(adj_ref, y_ref, b_ref, wn_ref, bn_ref, o_ref, *, out_f32):
    a_bf = adj_ref[...].astype(jnp.bfloat16)
    acc = jnp.dot(a_bf, y_ref[...], preferred_element_type=jnp.float32)
    h = jnp.maximum(acc + b_ref[...], 0.0).astype(jnp.bfloat16)
    r = jnp.dot(h, wn_ref[...], preferred_element_type=jnp.float32)
    if out_f32:
        o_ref[...] = r + bn_ref[...]
    else:
        o_ref[...] = (r * (1.0 / 127.0)).astype(jnp.bfloat16)


def _fused_layer(adj_q, y, b, wn, bn, out_f32, off=0, rows=None):
    n = adj_q.shape[1]
    rows = adj_q.shape[0] if rows is None else rows
    kdim = y.shape[1]
    ow = wn.shape[1]
    return pl.pallas_call(
        functools.partial(_layer_body, out_f32=out_f32),
        grid=(rows // _BM2,),
        in_specs=[
            pl.BlockSpec((_BM2, n), lambda i: (i + off, 0)),
            pl.BlockSpec((n, kdim), lambda i: (0, 0)),
            pl.BlockSpec((1, kdim), lambda i: (0, 0)),
            pl.BlockSpec((kdim, ow), lambda i: (0, 0)),
            pl.BlockSpec((1, ow), lambda i: (0, 0)),
        ],
        out_specs=pl.BlockSpec((_BM2, ow), lambda i: (i, 0)),
        out_shape=jax.ShapeDtypeStruct(
            (rows, ow), jnp.float32 if out_f32 else jnp.bfloat16),
        compiler_params=pltpu.CompilerParams(
            dimension_semantics=("parallel",)),
    )(adj_q, y, b, wn, bn)


def _make_seg_sum(cpt, chunk):
    rows_per_tile = cpt * chunk
    mesh = plsc.VectorSubcoreMesh(core_axis_name="c", subcore_axis_name="s")

    @functools.partial(
        pl.kernel,
        mesh=mesh,
        out_type=jax.ShapeDtypeStruct((_TILES, _N_SEG, _N_SEG), jnp.float32),
        scratch_types=[
            pltpu.VMEM((rows_per_tile, _N_SEG), jnp.float32),
            pltpu.VMEM((cpt, chunk), jnp.int32),
            pltpu.VMEM((_N_SEG, _N_SEG), jnp.float32),
        ],
    )
    def seg_sum(x_hbm, idx_hbm, out_hbm, rows_v, idx_v, acc_v):
        cid = lax.axis_index("c")
        sid = lax.axis_index("s")
        wid = sid * 2 + cid
        pltpu.sync_copy(
            x_hbm.at[pl.ds(wid * rows_per_tile, rows_per_tile)], rows_v)
        pltpu.sync_copy(idx_hbm.at[wid], idx_v)

        zero = jnp.zeros((16,), jnp.float32)
        for r in range(_N_SEG):
            for j in range(_N_SEG // 16):
                acc_v[r, pl.ds(j * 16, 16)] = zero

        for c in range(cpt):
            for g in range(chunk // 16):
                svec = idx_v[c, pl.ds(g * 16, 16)]
                for k in range(16):
                    s = svec[k]
                    r = c * chunk + g * 16 + k
                    for j in range(_N_SEG // 16):
                        plsc.addupdate(acc_v.at[s, pl.ds(j * 16, 16)],
                                       rows_v[r, pl.ds(j * 16, 16)])

        pltpu.sync_copy(acc_v, out_hbm.at[wid])

    return seg_sum


_HALF = _N // 2
_CHUNK_H = 96
_CPT_H = 2
_RPT_H = _CHUNK_H * _CPT_H
_N_PAD_H = _TILES * _RPT_H
_seg_sum_half = _make_seg_sum(_CPT_H, _CHUNK_H)


def _tail_body(pa_ref, pb_ref, o_ref):
    p = jnp.sum(pa_ref[...], axis=0) + jnp.sum(pb_ref[...], axis=0)
    m = jnp.max(p, axis=1, keepdims=True)
    s = jnp.sum(jnp.exp(p - m), axis=1, keepdims=True)
    o_ref[...] = (p - m) - jnp.log(s)


def _tail(pa, pb):
    spec = pl.BlockSpec((_TILES, _N_SEG, _N_SEG), lambda: (0, 0, 0))
    return pl.pallas_call(
        _tail_body,
        in_specs=[spec, spec],
        out_specs=pl.BlockSpec((_N_SEG, _N_SEG), lambda: (0, 0)),
        out_shape=jax.ShapeDtypeStruct((_N_SEG, _N_SEG), jnp.float32),
    )(pa, pb)


def kernel(x_in, adj, idx, W1, b1, W2, b2, W3, b3, W4, b4):
    bf16 = jnp.bfloat16
    y2, adj_q = _layer1(x_in, W1, adj, b1.reshape(1, -1), W2.astype(bf16))
    y3 = _fused_layer(adj_q, y2, b2.reshape(1, -1), W3.astype(bf16),
                      jnp.zeros((1, W3.shape[1]), jnp.float32), False)
    w4 = W4.astype(bf16)
    b3r = b3.reshape(1, -1)
    b4r = b4.reshape(1, -1)
    idx32 = idx.astype(jnp.int32)
    x4a = _fused_layer(adj_q, y3, b3r, w4, b4r, True,
                       off=0, rows=_HALF)
    x4b = _fused_layer(adj_q, y3, b3r, w4, b4r, True,
                       off=_HALF // _BM2, rows=_HALF)

    pad_h = _N_PAD_H - _HALF
    xa = jnp.pad(x4a, ((0, pad_h), (0, 0)))
    xb = jnp.pad(x4b, ((0, pad_h), (0, 0)))
    ia = jnp.pad(idx32[:_HALF], (0, pad_h)).reshape(_TILES, _CPT_H, _CHUNK_H)
    ib = jnp.pad(idx32[_HALF:], (0, pad_h)).reshape(_TILES, _CPT_H, _CHUNK_H)
    parts_a = _seg_sum_half(xa, ia)
    parts_b = _seg_sum_half(xb, ib)
    return _tail(parts_a, parts_b)


# speedup vs baseline: 1.0115x; 1.0115x over previous
import functools

import jax
import jax.numpy as jnp
from jax import lax
from jax.experimental import pallas as pl
from jax.experimental.pallas import tpu as pltpu
from jax.experimental.pallas import tpu_sc as plsc

_N = 10000
_N_SEG = 64
_BM = 200

_TILES = 32
_CHUNK = 128
_CPT = 3
_ROWS_PER_TILE = _CHUNK * _CPT
_N_PAD = _TILES * _ROWS_PER_TILE


def _layer1_body(x_ref, w1_ref, adj_ref, b_ref, wn_ref, o_ref, adj_q_ref,
                 y1_ref):
    @pl.when(pl.program_id(0) == 0)
    def _():
        y1_ref[...] = jnp.dot(
            x_ref[...], w1_ref[...],
            preferred_element_type=jnp.float32).astype(jnp.bfloat16)

    a = adj_ref[...]
    adj_q_ref[...] = jnp.round(a * 127.0).astype(jnp.int8)
    acc = jnp.dot(a.astype(jnp.bfloat16), y1_ref[...],
                  preferred_element_type=jnp.float32)
    h = jnp.maximum(acc + b_ref[...], 0.0).astype(jnp.bfloat16)
    o_ref[...] = (jnp.dot(h, wn_ref[...], preferred_element_type=jnp.float32)
                  * (1.0 / 127.0)).astype(jnp.bfloat16)


def _layer1(x_in, W1, adj, b, wn):
    n = adj.shape[0]
    d = x_in.shape[1]
    kdim = W1.shape[1]
    ow = wn.shape[1]
    return pl.pallas_call(
        _layer1_body,
        grid=(n // _BM,),
        in_specs=[
            pl.BlockSpec((n, d), lambda i: (0, 0)),
            pl.BlockSpec((d, kdim), lambda i: (0, 0)),
            pl.BlockSpec((_BM, n), lambda i: (i, 0)),
            pl.BlockSpec((1, kdim), lambda i: (0, 0)),
            pl.BlockSpec((kdim, ow), lambda i: (0, 0)),
        ],
        out_specs=[
            pl.BlockSpec((_BM, ow), lambda i: (i, 0)),
            pl.BlockSpec((_BM, n), lambda i: (i, 0)),
        ],
        out_shape=[
            jax.ShapeDtypeStruct((n, ow), jnp.bfloat16),
            jax.ShapeDtypeStruct((n, n), jnp.int8),
        ],
        scratch_shapes=[pltpu.VMEM((n, kdim), jnp.bfloat16)],
        compiler_params=pltpu.CompilerParams(
            dimension_semantics=("arbitrary",)),
    )(x_in, W1, adj, b, wn)


_BM2 = 1000


def _layer_body(adj_ref, y_ref, b_ref, wn_ref, bn_ref, o_ref, *, out_f32):
    a_bf = adj_ref[...].astype(jnp.bfloat16)
    acc = jnp.dot(a_bf, y_ref[...], preferred_element_type=jnp.float32)
    h = jnp.maximum(acc + b_ref[...], 0.0).astype(jnp.bfloat16)
    r = jnp.dot(h, wn_ref[...], preferred_element_type=jnp.float32)
    if out_f32:
        o_ref[...] = r + bn_ref[...]
    else:
        o_ref[...] = (r * (1.0 / 127.0)).astype(jnp.bfloat16)


def _fused_layer(adj_q, y, b, wn, bn, out_f32):
    n = adj_q.shape[0]
    kdim = y.shape[1]
    ow = wn.shape[1]
    return pl.pallas_call(
        functools.partial(_layer_body, out_f32=out_f32),
        grid=(n // _BM2,),
        in_specs=[
            pl.BlockSpec((_BM2, n), lambda i: (i, 0)),
            pl.BlockSpec((n, kdim), lambda i: (0, 0)),
            pl.BlockSpec((1, kdim), lambda i: (0, 0)),
            pl.BlockSpec((kdim, ow), lambda i: (0, 0)),
            pl.BlockSpec((1, ow), lambda i: (0, 0)),
        ],
        out_specs=pl.BlockSpec((_BM2, ow), lambda i: (i, 0)),
        out_shape=jax.ShapeDtypeStruct(
            (n, ow), jnp.float32 if out_f32 else jnp.bfloat16),
        compiler_params=pltpu.CompilerParams(
            dimension_semantics=("parallel",)),
    )(adj_q, y, b, wn, bn)


def _make_seg_sum():
    mesh = plsc.VectorSubcoreMesh(core_axis_name="c", subcore_axis_name="s")

    @functools.partial(
        pl.kernel,
        mesh=mesh,
        out_type=jax.ShapeDtypeStruct((_TILES, _N_SEG, _N_SEG), jnp.float32),
        scratch_types=[
            pltpu.VMEM((_ROWS_PER_TILE, _N_SEG), jnp.float32),
            pltpu.VMEM((_CPT, _CHUNK), jnp.int32),
            pltpu.VMEM((_N_SEG, _N_SEG), jnp.float32),
        ],
    )
    def seg_sum(x_hbm, idx_hbm, out_hbm, rows_v, idx_v, acc_v):
        cid = lax.axis_index("c")
        sid = lax.axis_index("s")
        wid = sid * 2 + cid
        pltpu.sync_copy(
            x_hbm.at[pl.ds(wid * _ROWS_PER_TILE, _ROWS_PER_TILE)], rows_v)
        pltpu.sync_copy(idx_hbm.at[wid], idx_v)

        zero = jnp.zeros((16,), jnp.float32)
        for r in range(_N_SEG):
            for j in range(_N_SEG // 16):
                acc_v[r, pl.ds(j * 16, 16)] = zero

        for c in range(_CPT):
            for g in range(_CHUNK // 16):
                svec = idx_v[c, pl.ds(g * 16, 16)]
                for k in range(16):
                    s = svec[k]
                    r = c * _CHUNK + g * 16 + k
                    for j in range(_N_SEG // 16):
                        plsc.addupdate(acc_v.at[s, pl.ds(j * 16, 16)],
                                       rows_v[r, pl.ds(j * 16, 16)])

        pltpu.sync_copy(acc_v, out_hbm.at[wid])

    return seg_sum


_seg_sum = _make_seg_sum()


def _tail_body(p_ref, o_ref):
    p = jnp.sum(p_ref[...], axis=0)
    m = jnp.max(p, axis=1, keepdims=True)
    s = jnp.sum(jnp.exp(p - m), axis=1, keepdims=True)
    o_ref[...] = (p - m) - jnp.log(s)


def _tail(parts):
    return pl.pallas_call(
        _tail_body,
        in_specs=[pl.BlockSpec((_TILES, _N_SEG, _N_SEG),
                               lambda: (0, 0, 0))],
        out_specs=pl.BlockSpec((_N_SEG, _N_SEG), lambda: (0, 0)),
        out_shape=jax.ShapeDtypeStruct((_N_SEG, _N_SEG), jnp.float32),
    )(parts)


def kernel(x_in, adj, idx, W1, b1, W2, b2, W3, b3, W4, b4):
    bf16 = jnp.bfloat16
    y2, adj_q = _layer1(x_in, W1, adj, b1.reshape(1, -1), W2.astype(bf16))
    y3 = _fused_layer(adj_q, y2, b2.reshape(1, -1), W3.astype(bf16),
                      jnp.zeros((1, W3.shape[1]), jnp.float32), False)
    x4 = _fused_layer(adj_q, y3, b3.reshape(1, -1), W4.astype(bf16),
                      b4.reshape(1, -1), True)

    x4p = jnp.pad(x4, ((0, _N_PAD - _N), (0, 0)))
    idxp = jnp.pad(idx.astype(jnp.int32), (0, _N_PAD - _N))
    idxp = idxp.reshape(_TILES, _CPT, _CHUNK)
    parts = _seg_sum(x4p, idxp)
    return _tail(parts)
